# Initial kernel scaffold; baseline (speedup 1.0000x reference)
import jax
import jax.numpy as jnp
from jax.experimental import pallas as pl


def kernel(xn, xe, edge_index, K1Nopen, K2Nopen, K1Eopen, K2Eopen, KNclose, KE1, KE2, KN1, KN2):
    raise NotImplementedError("write your pallas kernel here")


# speedup vs baseline: 3.0665x; 3.0665x over previous
import functools

import jax
import jax.numpy as jnp
from jax import lax
from jax.experimental import pallas as pl
from jax.experimental.pallas import tpu as pltpu
from jax.experimental.pallas import tpu_sc as plsc

N = 10000
E = 320000
C = 32
NLAYER = 4
H = 0.1
EPS = 1e-5
TILE_E = 8000
GRID_E = E // TILE_E


def _stats_from(st_ref, tot):
    m = st_ref[0] / tot
    v = st_ref[1] / tot - m * m
    return m, jax.lax.rsqrt(v + EPS)


def _node_open_body(xn_ref, k1t_ref, k2t_ref, wp_ref, wq_ref,
                    xn0_ref, p_ref, q_ref):
    t = jnp.dot(xn_ref[...], k1t_ref[...], preferred_element_type=jnp.float32)
    m = jnp.mean(t)
    v = jnp.mean((t - m) ** 2)
    z = jnp.maximum((t - m) * jax.lax.rsqrt(v + EPS), 0.0)
    xn0 = jnp.dot(z, k2t_ref[...], preferred_element_type=jnp.float32)
    xn0_ref[...] = xn0
    p_ref[...] = jnp.dot(xn0, wp_ref[...], preferred_element_type=jnp.float32)
    q_ref[...] = jnp.dot(xn0, wq_ref[...], preferred_element_type=jnp.float32)


def _node_layer_body(sd_ref, ss_ref, xn_ref, wsd_ref, wss_ref, wx_ref,
                     kn2t_ref, wp_ref, wq_ref, xn1_ref, p_ref, q_ref):
    sd = sd_ref[0] + sd_ref[1]
    ss = ss_ref[0] + ss_ref[1]
    yn = (jnp.dot(sd, wsd_ref[...], preferred_element_type=jnp.float32)
          + jnp.dot(ss, wss_ref[...], preferred_element_type=jnp.float32)
          + jnp.dot(xn_ref[...], wx_ref[...], preferred_element_type=jnp.float32))
    m = jnp.mean(yn)
    v = jnp.mean((yn - m) ** 2)
    z = jnp.maximum((yn - m) * jax.lax.rsqrt(v + EPS), 0.0)
    xn1 = xn_ref[...] + H * jnp.dot(z, kn2t_ref[...],
                                    preferred_element_type=jnp.float32)
    xn1_ref[...] = xn1
    p_ref[...] = jnp.dot(xn1, wp_ref[...], preferred_element_type=jnp.float32)
    q_ref[...] = jnp.dot(xn1, wq_ref[...], preferred_element_type=jnp.float32)


def _edge_open1_body(xe_ref, w_ref, t_ref, st_ref):
    i = pl.program_id(0)
    t = jnp.dot(xe_ref[...], w_ref[...], preferred_element_type=jnp.float32)
    t_ref[...] = t

    @pl.when(i == 0)
    def _():
        st_ref[0] = 0.0
        st_ref[1] = 0.0

    st_ref[0] += jnp.sum(t)
    st_ref[1] += jnp.sum(t * t)


def _edge_open2_body(t_ref, st_ref, k2t_ref, xe_ref):
    m, rs = _stats_from(st_ref, float(C * E))
    z = jnp.maximum((t_ref[...] - m) * rs, 0.0)
    xe_ref[...] = jnp.dot(z, k2t_ref[...], preferred_element_type=jnp.float32)


def _edge_k1_body(g_ref, xe_ref, wb_ref, y1_ref, st_ref):
    i = pl.program_id(0)
    y1 = g_ref[...] + jnp.dot(xe_ref[...], wb_ref[...],
                              preferred_element_type=jnp.float32)
    y1_ref[...] = y1

    @pl.when(i == 0)
    def _():
        st_ref[0] = 0.0
        st_ref[1] = 0.0

    st_ref[0] += jnp.sum(y1)
    st_ref[1] += jnp.sum(y1 * y1)


def _edge_k2_body(y1_ref, st1_ref, ke2t_ref, st3_ref):
    i = pl.program_id(0)
    m, rs = _stats_from(st1_ref, float(C * E))
    z = jnp.maximum((y1_ref[...] - m) * rs, 0.0)
    y3 = jnp.dot(z, ke2t_ref[...], preferred_element_type=jnp.float32)

    @pl.when(i == 0)
    def _():
        st3_ref[0] = 0.0
        st3_ref[1] = 0.0

    st3_ref[0] += jnp.sum(y3)
    st3_ref[1] += jnp.sum(y3 * y3)


def _edge_k3_body(y1_ref, xe_ref, st1_ref, st3_ref, ke2t_ref, xe1_ref):
    m1, rs1 = _stats_from(st1_ref, float(C * E))
    z = jnp.maximum((y1_ref[...] - m1) * rs1, 0.0)
    y3 = jnp.dot(z, ke2t_ref[...], preferred_element_type=jnp.float32)
    m3, rs3 = _stats_from(st3_ref, float(C * E))
    xe1_ref[...] = xe_ref[...] + H * ((y3 - m3) * rs3)


def _full(shape):
    return pl.BlockSpec(shape, lambda i: tuple(0 for _ in shape))


def _node_open(xn_r, k1t, k2t, wp, wq):
    return pl.pallas_call(
        _node_open_body,
        out_shape=[jax.ShapeDtypeStruct((N, C), jnp.float32)] * 3,
    )(xn_r, k1t, k2t, wp, wq)


def _node_layer(sd, ss, xn_r, wsd, wss, wx, kn2t, wp, wq):
    return pl.pallas_call(
        _node_layer_body,
        out_shape=[jax.ShapeDtypeStruct((N, C), jnp.float32)] * 3,
    )(sd, ss, xn_r, wsd, wss, wx, kn2t, wp, wq)


def _edge_open(xe_r, k1t, k2t):
    t, st = pl.pallas_call(
        _edge_open1_body,
        grid=(GRID_E,),
        in_specs=[pl.BlockSpec((TILE_E, 16), lambda i: (i, 0)),
                  _full((16, C))],
        out_specs=[pl.BlockSpec((TILE_E, C), lambda i: (i, 0)),
                   pl.BlockSpec(memory_space=pltpu.SMEM)],
        out_shape=[jax.ShapeDtypeStruct((E, C), jnp.float32),
                   jax.ShapeDtypeStruct((2,), jnp.float32)],
    )(xe_r, k1t)
    return pl.pallas_call(
        _edge_open2_body,
        grid=(GRID_E,),
        in_specs=[pl.BlockSpec((TILE_E, C), lambda i: (i, 0)),
                  pl.BlockSpec(memory_space=pltpu.SMEM),
                  _full((C, C))],
        out_specs=pl.BlockSpec((TILE_E, C), lambda i: (i, 0)),
        out_shape=jax.ShapeDtypeStruct((E, C), jnp.float32),
    )(t, st, k2t)


def _edge_path(g, xe_r, wb, ke2t):
    y1, st1 = pl.pallas_call(
        _edge_k1_body,
        grid=(GRID_E,),
        in_specs=[pl.BlockSpec((TILE_E, C), lambda i: (i, 0)),
                  pl.BlockSpec((TILE_E, C), lambda i: (i, 0)),
                  _full((C, C))],
        out_specs=[pl.BlockSpec((TILE_E, C), lambda i: (i, 0)),
                   pl.BlockSpec(memory_space=pltpu.SMEM)],
        out_shape=[jax.ShapeDtypeStruct((E, C), jnp.float32),
                   jax.ShapeDtypeStruct((2,), jnp.float32)],
    )(g, xe_r, wb)
    st3 = pl.pallas_call(
        _edge_k2_body,
        grid=(GRID_E,),
        in_specs=[pl.BlockSpec((TILE_E, C), lambda i: (i, 0)),
                  pl.BlockSpec(memory_space=pltpu.SMEM),
                  _full((C, C))],
        out_specs=pl.BlockSpec(memory_space=pltpu.SMEM),
        out_shape=jax.ShapeDtypeStruct((2,), jnp.float32),
    )(y1, st1, ke2t)
    xe1 = pl.pallas_call(
        _edge_k3_body,
        grid=(GRID_E,),
        in_specs=[pl.BlockSpec((TILE_E, C), lambda i: (i, 0)),
                  pl.BlockSpec((TILE_E, C), lambda i: (i, 0)),
                  pl.BlockSpec(memory_space=pltpu.SMEM),
                  pl.BlockSpec(memory_space=pltpu.SMEM),
                  _full((C, C))],
        out_specs=pl.BlockSpec((TILE_E, C), lambda i: (i, 0)),
        out_shape=jax.ShapeDtypeStruct((E, C), jnp.float32),
    )(y1, xe_r, st1, st3, ke2t)
    return xe1


SC_NC = 2
SC_NS = 16
SC_NW = SC_NC * SC_NS
EW = E // SC_NW
CH = 80
NCH = EW // CH
ZT = 10
ZROW = N // ZT


def _sc_graph_body(p_hbm, q_hbm, xe_hbm, dst_hbm, src_hbm, zero_hbm,
                   g_hbm, sdp_hbm, ssp_hbm,
                   idxd_v, idxs_v, bufp_v, bufq_v, bufe_v,
                   sd_acc, ss_acc, semp, semq):
    cid = lax.axis_index("c")
    sid = lax.axis_index("s")
    wid = sid * SC_NC + cid

    pltpu.sync_copy(dst_hbm.at[wid], idxd_v)
    pltpu.sync_copy(src_hbm.at[wid], idxs_v)

    zs = pl.ds(sid * ZROW, ZROW)

    @pl.when(sid < ZT)
    def _():
        pltpu.sync_copy(zero_hbm.at[zs], sd_acc.at[zs])
        pltpu.sync_copy(zero_hbm.at[zs], ss_acc.at[zs])

    plsc.subcore_barrier()

    def chunk(c, carry):
        r = wid * NCH + c
        cpp = pltpu.async_copy(p_hbm.at[idxd_v.at[c]], bufp_v, semp)
        cpq = pltpu.async_copy(q_hbm.at[idxs_v.at[c]], bufq_v, semq)
        pltpu.sync_copy(xe_hbm.at[r], bufe_v)
        cpp.wait()
        cpq.wait()

        def add_row(i, carry2):
            for j in range(C // 16):
                s = pl.ds(j * 16, 16)
                bufp_v[i, s] = bufp_v[i, s] + bufq_v[i, s]
            return carry2

        lax.fori_loop(0, CH, add_row, 0, unroll=4)
        pltpu.sync_copy(bufp_v, g_hbm.at[r])
        pltpu.sync_copy(bufe_v, sd_acc.at[idxd_v.at[c]], add=True)
        pltpu.sync_copy(bufe_v, ss_acc.at[idxs_v.at[c]], add=True)
        return carry

    lax.fori_loop(0, NCH, chunk, 0)
    plsc.subcore_barrier()

    @pl.when(sid < ZT)
    def _():
        pltpu.sync_copy(sd_acc.at[zs], sdp_hbm.at[cid, zs])
        pltpu.sync_copy(ss_acc.at[zs], ssp_hbm.at[cid, zs])


def _graph_ops_sc(p, q, xe3, dst3d, src3d, zero):
    return pl.kernel(
        _sc_graph_body,
        out_type=[jax.ShapeDtypeStruct((E // CH, CH, C), jnp.float32),
                  jax.ShapeDtypeStruct((SC_NC, N, C), jnp.float32),
                  jax.ShapeDtypeStruct((SC_NC, N, C), jnp.float32)],
        mesh=plsc.VectorSubcoreMesh(core_axis_name="c", subcore_axis_name="s",
                                    num_cores=SC_NC, num_subcores=SC_NS),
        compiler_params=pltpu.CompilerParams(use_tc_tiling_on_sc=False),
        scratch_types=[
            pltpu.VMEM((NCH, CH), jnp.int32),
            pltpu.VMEM((NCH, CH), jnp.int32),
            pltpu.VMEM((CH, C), jnp.float32),
            pltpu.VMEM((CH, C), jnp.float32),
            pltpu.VMEM((CH, C), jnp.float32),
            pltpu.VMEM_SHARED((N, C), jnp.float32),
            pltpu.VMEM_SHARED((N, C), jnp.float32),
            pltpu.SemaphoreType.DMA,
            pltpu.SemaphoreType.DMA,
        ],
    )(p, q, xe3, dst3d, src3d, zero)


def _graph_ops(p, q, xe_r, dst3d, src3d, zero):
    g3, sdp, ssp = _graph_ops_sc(p, q, xe_r.reshape(E // CH, CH, C),
                                 dst3d, src3d, zero)
    return g3.reshape(E, C), sdp, ssp


def kernel(xn, xe, edge_index, K1Nopen, K2Nopen, K1Eopen, K2Eopen, KNclose,
           KE1, KE2, KN1, KN2):
    xn_r = xn[0].T
    xe_r = xe[0].T
    src3d = edge_index[0].reshape(SC_NW, NCH, CH)
    dst3d = edge_index[1].reshape(SC_NW, NCH, CH)
    zero = jnp.zeros((N, C), jnp.float32)

    A, B, Cm = KE1[:, :, 0:C], KE1[:, :, C:2 * C], KE1[:, :, 2 * C:3 * C]
    WP = jnp.transpose(0.5 * A + Cm, (0, 2, 1))
    WQ = jnp.transpose(0.5 * A - Cm, (0, 2, 1))
    WB = jnp.transpose(B, (0, 2, 1))
    KE2T = jnp.transpose(KE2, (0, 2, 1))
    D, F, Gm = KN1[:, :, 0:C], KN1[:, :, C:2 * C], KN1[:, :, 2 * C:3 * C]
    WSD = jnp.transpose(0.5 * D + F, (0, 2, 1))
    WSS = jnp.transpose(0.5 * D - F, (0, 2, 1))
    WX = jnp.transpose(Gm, (0, 2, 1))
    KN2T = jnp.transpose(KN2, (0, 2, 1))
    KNcloseT = KNclose.T

    xn_r, p, q = _node_open(xn_r, K1Nopen.T, K2Nopen.T, WP[0], WQ[0])
    xe_r = _edge_open(xe_r, K1Eopen.T, K2Eopen.T)

    for i in range(NLAYER):
        g, sdp, ssp = _graph_ops(p, q, xe_r, dst3d, src3d, zero)
        xe_r = _edge_path(g, xe_r, WB[i], KE2T[i])
        if i + 1 < NLAYER:
            wp_n, wq_n = WP[i + 1], WQ[i + 1]
        else:
            wp_n, wq_n = KNcloseT, KNcloseT
        xn_r, p, q = _node_layer(sdp, ssp, xn_r, WSD[i], WSS[i], WX[i],
                                 KN2T[i], wp_n, wq_n)

    return (p.T[None], xe_r.T[None])


# speedup vs baseline: 6.0394x; 1.9695x over previous
import functools

import jax
import jax.numpy as jnp
from jax import lax
from jax.experimental import pallas as pl
from jax.experimental.pallas import tpu as pltpu
from jax.experimental.pallas import tpu_sc as plsc

N = 10000
E = 320000
C = 32
NLAYER = 4
H = 0.1
EPS = 1e-5
E4 = E // 4
TILE_E = 8000
GRID_E = E4 // TILE_E


def _stats_from(st_ref, tot):
    m = st_ref[0] / tot
    v = st_ref[1] / tot - m * m
    return m, jax.lax.rsqrt(v + EPS)


def _node_open_body(xn_ref, k1t_ref, k2t_ref, wp_ref, wq_ref,
                    xn0_ref, p_ref, q_ref):
    t = jnp.dot(xn_ref[...], k1t_ref[...], preferred_element_type=jnp.float32)
    m = jnp.mean(t)
    v = jnp.mean((t - m) ** 2)
    z = jnp.maximum((t - m) * jax.lax.rsqrt(v + EPS), 0.0)
    xn0 = jnp.dot(z, k2t_ref[...], preferred_element_type=jnp.float32)
    xn0_ref[...] = xn0
    p_ref[...] = jnp.dot(xn0, wp_ref[...], preferred_element_type=jnp.float32)
    q_ref[...] = jnp.dot(xn0, wq_ref[...], preferred_element_type=jnp.float32)


def _node_layer_body(sd_ref, ss_ref, xn_ref, wsd_ref, wss_ref, wx_ref,
                     kn2t_ref, wp_ref, wq_ref, xn1_ref, p_ref, q_ref):
    sd = sd_ref[0] + sd_ref[1]
    ss = ss_ref[0] + ss_ref[1]
    yn = (jnp.dot(sd, wsd_ref[...], preferred_element_type=jnp.float32)
          + jnp.dot(ss, wss_ref[...], preferred_element_type=jnp.float32)
          + jnp.dot(xn_ref[...], wx_ref[...], preferred_element_type=jnp.float32))
    m = jnp.mean(yn)
    v = jnp.mean((yn - m) ** 2)
    z = jnp.maximum((yn - m) * jax.lax.rsqrt(v + EPS), 0.0)
    xn1 = xn_ref[...] + H * jnp.dot(z, kn2t_ref[...],
                                    preferred_element_type=jnp.float32)
    xn1_ref[...] = xn1
    p_ref[...] = jnp.dot(xn1, wp_ref[...], preferred_element_type=jnp.float32)
    q_ref[...] = jnp.dot(xn1, wq_ref[...], preferred_element_type=jnp.float32)


def _edge_open1_body(xe_ref, w_ref, t_ref, st_ref):
    i = pl.program_id(0)
    t = jnp.dot(xe_ref[...], w_ref[...], preferred_element_type=jnp.float32)
    t_ref[...] = t

    @pl.when(i == 0)
    def _():
        st_ref[0] = 0.0
        st_ref[1] = 0.0

    st_ref[0] += jnp.sum(t)
    st_ref[1] += jnp.sum(t * t)


def _edge_open2_body(t_ref, st_ref, k2t_ref, xe_ref):
    m, rs = _stats_from(st_ref, float(C * E))
    z = jnp.maximum((t_ref[...] - m) * rs, 0.0)
    xe_ref[...] = jnp.dot(z, k2t_ref[...], preferred_element_type=jnp.float32)


def _edge_k1_body(g_ref, xe_ref, wb_ref, y1_ref, st_ref):
    i = pl.program_id(0)
    y1 = g_ref[...] + jnp.dot(xe_ref[...], wb_ref[...],
                              preferred_element_type=jnp.float32)
    y1_ref[...] = y1

    @pl.when(i == 0)
    def _():
        st_ref[0] = 0.0
        st_ref[1] = 0.0

    st_ref[0] += jnp.sum(y1)
    st_ref[1] += jnp.sum(y1 * y1)


def _edge_k2_body(y1_ref, st1_ref, ke2t_ref, st3_ref):
    i = pl.program_id(0)
    m, rs = _stats_from(st1_ref, float(C * E))
    z = jnp.maximum((y1_ref[...] - m) * rs, 0.0)
    y3 = jnp.dot(z, ke2t_ref[...], preferred_element_type=jnp.float32)

    @pl.when(i == 0)
    def _():
        st3_ref[0] = 0.0
        st3_ref[1] = 0.0

    st3_ref[0] += jnp.sum(y3)
    st3_ref[1] += jnp.sum(y3 * y3)


def _edge_k3_body(y1_ref, xe_ref, st1_ref, st3_ref, ke2t_ref, xe1_ref):
    m1, rs1 = _stats_from(st1_ref, float(C * E))
    z = jnp.maximum((y1_ref[...] - m1) * rs1, 0.0)
    y3 = jnp.dot(z, ke2t_ref[...], preferred_element_type=jnp.float32)
    m3, rs3 = _stats_from(st3_ref, float(C * E))
    xe1_ref[...] = xe_ref[...] + H * ((y3 - m3) * rs3)


def _full(shape):
    return pl.BlockSpec(shape, lambda i: tuple(0 for _ in shape))


def _node_open(xn_r, k1t, k2t, wp, wq):
    return pl.pallas_call(
        _node_open_body,
        out_shape=[jax.ShapeDtypeStruct((N, C), jnp.float32)] * 3,
    )(xn_r, k1t, k2t, wp, wq)


def _node_layer(sd, ss, xn_r, wsd, wss, wx, kn2t, wp, wq):
    return pl.pallas_call(
        _node_layer_body,
        out_shape=[jax.ShapeDtypeStruct((N, C), jnp.float32)] * 3,
    )(sd, ss, xn_r, wsd, wss, wx, kn2t, wp, wq)


def _edge_open(xe64, k1bd, k2bd):
    t, st = pl.pallas_call(
        _edge_open1_body,
        grid=(GRID_E,),
        in_specs=[pl.BlockSpec((TILE_E, 64), lambda i: (i, 0)),
                  _full((64, 128))],
        out_specs=[pl.BlockSpec((TILE_E, 128), lambda i: (i, 0)),
                   pl.BlockSpec(memory_space=pltpu.SMEM)],
        out_shape=[jax.ShapeDtypeStruct((E4, 128), jnp.float32),
                   jax.ShapeDtypeStruct((2,), jnp.float32)],
    )(xe64, k1bd)
    return pl.pallas_call(
        _edge_open2_body,
        grid=(GRID_E,),
        in_specs=[pl.BlockSpec((TILE_E, 128), lambda i: (i, 0)),
                  pl.BlockSpec(memory_space=pltpu.SMEM),
                  _full((128, 128))],
        out_specs=pl.BlockSpec((TILE_E, 128), lambda i: (i, 0)),
        out_shape=jax.ShapeDtypeStruct((E4, 128), jnp.float32),
    )(t, st, k2bd)


def _edge_path(g, xe128, wbbd, ke2bd):
    y1, st1 = pl.pallas_call(
        _edge_k1_body,
        grid=(GRID_E,),
        in_specs=[pl.BlockSpec((TILE_E, 128), lambda i: (i, 0)),
                  pl.BlockSpec((TILE_E, 128), lambda i: (i, 0)),
                  _full((128, 128))],
        out_specs=[pl.BlockSpec((TILE_E, 128), lambda i: (i, 0)),
                   pl.BlockSpec(memory_space=pltpu.SMEM)],
        out_shape=[jax.ShapeDtypeStruct((E4, 128), jnp.float32),
                   jax.ShapeDtypeStruct((2,), jnp.float32)],
    )(g, xe128, wbbd)
    st3 = pl.pallas_call(
        _edge_k2_body,
        grid=(GRID_E,),
        in_specs=[pl.BlockSpec((TILE_E, 128), lambda i: (i, 0)),
                  pl.BlockSpec(memory_space=pltpu.SMEM),
                  _full((128, 128))],
        out_specs=pl.BlockSpec(memory_space=pltpu.SMEM),
        out_shape=jax.ShapeDtypeStruct((2,), jnp.float32),
    )(y1, st1, ke2bd)
    xe1 = pl.pallas_call(
        _edge_k3_body,
        grid=(GRID_E,),
        in_specs=[pl.BlockSpec((TILE_E, 128), lambda i: (i, 0)),
                  pl.BlockSpec((TILE_E, 128), lambda i: (i, 0)),
                  pl.BlockSpec(memory_space=pltpu.SMEM),
                  pl.BlockSpec(memory_space=pltpu.SMEM),
                  _full((128, 128))],
        out_specs=pl.BlockSpec((TILE_E, 128), lambda i: (i, 0)),
        out_shape=jax.ShapeDtypeStruct((E4, 128), jnp.float32),
    )(y1, xe128, st1, st3, ke2bd)
    return xe1


SC_NC = 2
SC_NS = 16
SC_NW = SC_NC * SC_NS
EW = E // SC_NW
CH = 80
NCH = EW // CH
ZT = 10
ZROW = N // ZT


def _sc_graph_body(p_hbm, q_hbm, xe_hbm, dst_hbm, src_hbm, zero_hbm,
                   g_hbm, sdp_hbm, ssp_hbm,
                   idxd_v, idxs_v, bufp_v, bufq_v, bufe_v,
                   sd_acc, ss_acc, semp, semq):
    cid = lax.axis_index("c")
    sid = lax.axis_index("s")
    wid = sid * SC_NC + cid

    pltpu.sync_copy(dst_hbm.at[wid], idxd_v)
    pltpu.sync_copy(src_hbm.at[wid], idxs_v)

    zs = pl.ds(sid * ZROW, ZROW)

    @pl.when(sid < ZT)
    def _():
        pltpu.sync_copy(zero_hbm.at[zs], sd_acc.at[zs])
        pltpu.sync_copy(zero_hbm.at[zs], ss_acc.at[zs])

    plsc.subcore_barrier()

    def chunk(c, carry):
        r = wid * NCH + c
        cpp = pltpu.async_copy(p_hbm.at[idxd_v.at[c]], bufp_v, semp)
        cpq = pltpu.async_copy(q_hbm.at[idxs_v.at[c]], bufq_v, semq)
        pltpu.sync_copy(xe_hbm.at[r], bufe_v)
        cpp.wait()
        cpq.wait()

        def add_row(i, carry2):
            for j in range(C // 16):
                s = pl.ds(j * 16, 16)
                bufp_v[i, s] = bufp_v[i, s] + bufq_v[i, s]
            return carry2

        lax.fori_loop(0, CH, add_row, 0, unroll=4)
        pltpu.sync_copy(bufp_v, g_hbm.at[r])
        pltpu.sync_copy(bufe_v, sd_acc.at[idxd_v.at[c]], add=True)
        pltpu.sync_copy(bufe_v, ss_acc.at[idxs_v.at[c]], add=True)
        return carry

    lax.fori_loop(0, NCH, chunk, 0)
    plsc.subcore_barrier()

    @pl.when(sid < ZT)
    def _():
        pltpu.sync_copy(sd_acc.at[zs], sdp_hbm.at[cid, zs])
        pltpu.sync_copy(ss_acc.at[zs], ssp_hbm.at[cid, zs])


def _graph_ops_sc(p, q, xe3, dst3d, src3d, zero):
    return pl.kernel(
        _sc_graph_body,
        out_type=[jax.ShapeDtypeStruct((E // CH, CH, C), jnp.float32),
                  jax.ShapeDtypeStruct((SC_NC, N, C), jnp.float32),
                  jax.ShapeDtypeStruct((SC_NC, N, C), jnp.float32)],
        mesh=plsc.VectorSubcoreMesh(core_axis_name="c", subcore_axis_name="s",
                                    num_cores=SC_NC, num_subcores=SC_NS),
        compiler_params=pltpu.CompilerParams(use_tc_tiling_on_sc=False),
        scratch_types=[
            pltpu.VMEM((NCH, CH), jnp.int32),
            pltpu.VMEM((NCH, CH), jnp.int32),
            pltpu.VMEM((CH, C), jnp.float32),
            pltpu.VMEM((CH, C), jnp.float32),
            pltpu.VMEM((CH, C), jnp.float32),
            pltpu.VMEM_SHARED((N, C), jnp.float32),
            pltpu.VMEM_SHARED((N, C), jnp.float32),
            pltpu.SemaphoreType.DMA,
            pltpu.SemaphoreType.DMA,
        ],
    )(p, q, xe3, dst3d, src3d, zero)


def _graph_ops(p, q, xe128, dst3d, src3d, zero):
    g3, sdp, ssp = _graph_ops_sc(p, q, xe128.reshape(E // CH, CH, C),
                                 dst3d, src3d, zero)
    return g3.reshape(E4, 128), sdp, ssp


def kernel(xn, xe, edge_index, K1Nopen, K2Nopen, K1Eopen, K2Eopen, KNclose,
           KE1, KE2, KN1, KN2):
    xn_r = xn[0].T
    xe64 = xe[0].T.reshape(E4, 64)
    src3d = edge_index[0].reshape(SC_NW, NCH, CH)
    dst3d = edge_index[1].reshape(SC_NW, NCH, CH)
    zero = jnp.zeros((N, C), jnp.float32)

    i4 = jnp.eye(4, dtype=jnp.float32)
    A, B, Cm = KE1[:, :, 0:C], KE1[:, :, C:2 * C], KE1[:, :, 2 * C:3 * C]
    WP = jnp.transpose(0.5 * A + Cm, (0, 2, 1))
    WQ = jnp.transpose(0.5 * A - Cm, (0, 2, 1))
    WBbd = jnp.stack([jnp.kron(i4, jnp.transpose(B[i])) for i in range(NLAYER)])
    KE2bd = jnp.stack([jnp.kron(i4, jnp.transpose(KE2[i]))
                       for i in range(NLAYER)])
    D, F, Gm = KN1[:, :, 0:C], KN1[:, :, C:2 * C], KN1[:, :, 2 * C:3 * C]
    WSD = jnp.transpose(0.5 * D + F, (0, 2, 1))
    WSS = jnp.transpose(0.5 * D - F, (0, 2, 1))
    WX = jnp.transpose(Gm, (0, 2, 1))
    KN2T = jnp.transpose(KN2, (0, 2, 1))
    KNcloseT = KNclose.T

    xn_r, p, q = _node_open(xn_r, K1Nopen.T, K2Nopen.T, WP[0], WQ[0])
    xe128 = _edge_open(xe64, jnp.kron(i4, K1Eopen.T), jnp.kron(i4, K2Eopen.T))

    for i in range(NLAYER):
        g, sdp, ssp = _graph_ops(p, q, xe128, dst3d, src3d, zero)
        xe128 = _edge_path(g, xe128, WBbd[i], KE2bd[i])
        if i + 1 < NLAYER:
            wp_n, wq_n = WP[i + 1], WQ[i + 1]
        else:
            wp_n, wq_n = KNcloseT, KNcloseT
        xn_r, p, q = _node_layer(sdp, ssp, xn_r, WSD[i], WSS[i], WX[i],
                                 KN2T[i], wp_n, wq_n)

    return (p.T[None], xe128.reshape(E, C).T[None])


# speedup vs baseline: 8.3503x; 1.3826x over previous
import functools

import jax
import jax.numpy as jnp
from jax import lax
from jax.experimental import pallas as pl
from jax.experimental.pallas import tpu as pltpu
from jax.experimental.pallas import tpu_sc as plsc

N = 10000
E = 320000
C = 32
NLAYER = 4
H = 0.1
EPS = 1e-5
E4 = E // 4
TILE_E = 8000
GRID_E = E4 // TILE_E


def _stats_from(st_ref, tot):
    m = st_ref[0] / tot
    v = st_ref[1] / tot - m * m
    return m, jax.lax.rsqrt(v + EPS)


def _node_open_body(xn_ref, k1t_ref, k2t_ref, wp_ref, wq_ref,
                    xn0_ref, p_ref, q_ref):
    t = jnp.dot(xn_ref[...], k1t_ref[...], preferred_element_type=jnp.float32)
    m = jnp.mean(t)
    v = jnp.mean((t - m) ** 2)
    z = jnp.maximum((t - m) * jax.lax.rsqrt(v + EPS), 0.0)
    xn0 = jnp.dot(z, k2t_ref[...], preferred_element_type=jnp.float32)
    xn0_ref[...] = xn0
    p_ref[...] = jnp.dot(xn0, wp_ref[...], preferred_element_type=jnp.float32)
    q_ref[...] = jnp.dot(xn0, wq_ref[...], preferred_element_type=jnp.float32)


def _node_layer_body(sd_ref, ss_ref, xn_ref, wsd_ref, wss_ref, wx_ref,
                     kn2t_ref, wp_ref, wq_ref, xn1_ref, p_ref, q_ref):
    sd = sd_ref[0] + sd_ref[1]
    ss = ss_ref[0] + ss_ref[1]
    yn = (jnp.dot(sd, wsd_ref[...], preferred_element_type=jnp.float32)
          + jnp.dot(ss, wss_ref[...], preferred_element_type=jnp.float32)
          + jnp.dot(xn_ref[...], wx_ref[...], preferred_element_type=jnp.float32))
    m = jnp.mean(yn)
    v = jnp.mean((yn - m) ** 2)
    z = jnp.maximum((yn - m) * jax.lax.rsqrt(v + EPS), 0.0)
    xn1 = xn_ref[...] + H * jnp.dot(z, kn2t_ref[...],
                                    preferred_element_type=jnp.float32)
    xn1_ref[...] = xn1
    p_ref[...] = jnp.dot(xn1, wp_ref[...], preferred_element_type=jnp.float32)
    q_ref[...] = jnp.dot(xn1, wq_ref[...], preferred_element_type=jnp.float32)


def _edge_open1_body(xe_ref, w_ref, t_ref, st_ref):
    i = pl.program_id(0)
    t = jnp.dot(xe_ref[...], w_ref[...], preferred_element_type=jnp.float32)
    t_ref[...] = t

    @pl.when(i == 0)
    def _():
        st_ref[0] = 0.0
        st_ref[1] = 0.0

    st_ref[0] += jnp.sum(t)
    st_ref[1] += jnp.sum(t * t)


def _edge_open2_body(t_ref, st_ref, k2t_ref, xe_ref):
    m, rs = _stats_from(st_ref, float(C * E))
    z = jnp.maximum((t_ref[...] - m) * rs, 0.0)
    xe_ref[...] = jnp.dot(z, k2t_ref[...], preferred_element_type=jnp.float32)


def _edge_k1_body(g_ref, xe_ref, wb_ref, y1_ref, st_ref):
    i = pl.program_id(0)
    y1 = g_ref[...] + jnp.dot(xe_ref[...], wb_ref[...],
                              preferred_element_type=jnp.float32)
    y1_ref[...] = y1

    @pl.when(i == 0)
    def _():
        st_ref[0] = 0.0
        st_ref[1] = 0.0

    st_ref[0] += jnp.sum(y1)
    st_ref[1] += jnp.sum(y1 * y1)


def _edge_k2_body(y1_ref, st1_ref, ke2t_ref, st3_ref):
    i = pl.program_id(0)
    m, rs = _stats_from(st1_ref, float(C * E))
    z = jnp.maximum((y1_ref[...] - m) * rs, 0.0)
    y3 = jnp.dot(z, ke2t_ref[...], preferred_element_type=jnp.float32)

    @pl.when(i == 0)
    def _():
        st3_ref[0] = 0.0
        st3_ref[1] = 0.0

    st3_ref[0] += jnp.sum(y3)
    st3_ref[1] += jnp.sum(y3 * y3)


def _edge_k3_body(y1_ref, xe_ref, st1_ref, st3_ref, ke2t_ref, xe1_ref):
    m1, rs1 = _stats_from(st1_ref, float(C * E))
    z = jnp.maximum((y1_ref[...] - m1) * rs1, 0.0)
    y3 = jnp.dot(z, ke2t_ref[...], preferred_element_type=jnp.float32)
    m3, rs3 = _stats_from(st3_ref, float(C * E))
    xe1_ref[...] = xe_ref[...] + H * ((y3 - m3) * rs3)


def _full(shape):
    return pl.BlockSpec(shape, lambda i: tuple(0 for _ in shape))


def _node_open(xn_r, k1t, k2t, wp, wq):
    return pl.pallas_call(
        _node_open_body,
        out_shape=[jax.ShapeDtypeStruct((N, C), jnp.float32)] * 3,
    )(xn_r, k1t, k2t, wp, wq)


def _node_layer(sd, ss, xn_r, wsd, wss, wx, kn2t, wp, wq):
    return pl.pallas_call(
        _node_layer_body,
        out_shape=[jax.ShapeDtypeStruct((N, C), jnp.float32)] * 3,
    )(sd, ss, xn_r, wsd, wss, wx, kn2t, wp, wq)


def _edge_open(xe64, k1bd, k2bd):
    t, st = pl.pallas_call(
        _edge_open1_body,
        grid=(GRID_E,),
        in_specs=[pl.BlockSpec((TILE_E, 64), lambda i: (i, 0)),
                  _full((64, 128))],
        out_specs=[pl.BlockSpec((TILE_E, 128), lambda i: (i, 0)),
                   pl.BlockSpec(memory_space=pltpu.SMEM)],
        out_shape=[jax.ShapeDtypeStruct((E4, 128), jnp.float32),
                   jax.ShapeDtypeStruct((2,), jnp.float32)],
    )(xe64, k1bd)
    return pl.pallas_call(
        _edge_open2_body,
        grid=(GRID_E,),
        in_specs=[pl.BlockSpec((TILE_E, 128), lambda i: (i, 0)),
                  pl.BlockSpec(memory_space=pltpu.SMEM),
                  _full((128, 128))],
        out_specs=pl.BlockSpec((TILE_E, 128), lambda i: (i, 0)),
        out_shape=jax.ShapeDtypeStruct((E4, 128), jnp.float32),
    )(t, st, k2bd)


def _edge_path(g, xe128, wbbd, ke2bd):
    y1, st1 = pl.pallas_call(
        _edge_k1_body,
        grid=(GRID_E,),
        in_specs=[pl.BlockSpec((TILE_E, 128), lambda i: (i, 0)),
                  pl.BlockSpec((TILE_E, 128), lambda i: (i, 0)),
                  _full((128, 128))],
        out_specs=[pl.BlockSpec((TILE_E, 128), lambda i: (i, 0)),
                   pl.BlockSpec(memory_space=pltpu.SMEM)],
        out_shape=[jax.ShapeDtypeStruct((E4, 128), jnp.float32),
                   jax.ShapeDtypeStruct((2,), jnp.float32)],
    )(g, xe128, wbbd)
    st3 = pl.pallas_call(
        _edge_k2_body,
        grid=(GRID_E,),
        in_specs=[pl.BlockSpec((TILE_E, 128), lambda i: (i, 0)),
                  pl.BlockSpec(memory_space=pltpu.SMEM),
                  _full((128, 128))],
        out_specs=pl.BlockSpec(memory_space=pltpu.SMEM),
        out_shape=jax.ShapeDtypeStruct((2,), jnp.float32),
    )(y1, st1, ke2bd)
    xe1 = pl.pallas_call(
        _edge_k3_body,
        grid=(GRID_E,),
        in_specs=[pl.BlockSpec((TILE_E, 128), lambda i: (i, 0)),
                  pl.BlockSpec((TILE_E, 128), lambda i: (i, 0)),
                  pl.BlockSpec(memory_space=pltpu.SMEM),
                  pl.BlockSpec(memory_space=pltpu.SMEM),
                  _full((128, 128))],
        out_specs=pl.BlockSpec((TILE_E, 128), lambda i: (i, 0)),
        out_shape=jax.ShapeDtypeStruct((E4, 128), jnp.float32),
    )(y1, xe128, st1, st3, ke2bd)
    return xe1


SC_NC = 2
SC_NS = 16
SC_NW = SC_NC * SC_NS
EW = E // SC_NW
CH = 80
NCH = EW // CH
ZT = 10
ZROW = N // ZT


def _sc_gather_body(p_hbm, q_hbm, dst_hbm, src_hbm, g_hbm,
                    idxd_v, idxs_v, bufp_v, bufq_v, semp, semq):
    cid = lax.axis_index("c")
    sid = lax.axis_index("s")
    wid = sid * SC_NC + cid

    pltpu.sync_copy(dst_hbm.at[wid], idxd_v)
    pltpu.sync_copy(src_hbm.at[wid], idxs_v)

    def pair(c2, carry):
        c = 2 * c2
        r = wid * NCH + c
        cpp = pltpu.async_copy(p_hbm.at[idxd_v.at[c]], bufp_v, semp)
        cpq = pltpu.async_copy(p_hbm.at[idxd_v.at[c + 1]], bufq_v, semq)
        cpp.wait()
        cap = pltpu.async_copy(q_hbm.at[idxs_v.at[c]], bufp_v, semp, add=True)
        cpq.wait()
        caq = pltpu.async_copy(q_hbm.at[idxs_v.at[c + 1]], bufq_v, semq,
                               add=True)
        cap.wait()
        cop = pltpu.async_copy(bufp_v, g_hbm.at[r], semp)
        caq.wait()
        coq = pltpu.async_copy(bufq_v, g_hbm.at[r + 1], semq)
        cop.wait()
        coq.wait()
        return carry

    lax.fori_loop(0, NCH // 2, pair, 0)

    if NCH % 2:
        c = NCH - 1
        r = wid * NCH + c
        pltpu.async_copy(p_hbm.at[idxd_v.at[c]], bufp_v, semp).wait()
        pltpu.async_copy(q_hbm.at[idxs_v.at[c]], bufp_v, semp, add=True).wait()
        pltpu.sync_copy(bufp_v, g_hbm.at[r])


def _sc_scatter_body(xe_hbm, dst_hbm, src_hbm, zero_hbm,
                     sdp_hbm, ssp_hbm,
                     idxd_v, idxs_v, bufa_v, bufb_v,
                     sd_acc, ss_acc, sema, semb):
    cid = lax.axis_index("c")
    sid = lax.axis_index("s")
    wid = sid * SC_NC + cid

    pltpu.sync_copy(dst_hbm.at[wid], idxd_v)
    pltpu.sync_copy(src_hbm.at[wid], idxs_v)

    zs = pl.ds(sid * ZROW, ZROW)

    @pl.when(sid < ZT)
    def _():
        pltpu.sync_copy(zero_hbm.at[zs], sd_acc.at[zs])
        pltpu.sync_copy(zero_hbm.at[zs], ss_acc.at[zs])

    plsc.subcore_barrier()

    def pair(c2, carry):
        c = 2 * c2
        r = wid * NCH + c
        ca = pltpu.async_copy(xe_hbm.at[r], bufa_v, sema)
        cb = pltpu.async_copy(xe_hbm.at[r + 1], bufb_v, semb)
        ca.wait()
        pltpu.sync_copy(bufa_v, sd_acc.at[idxd_v.at[c]], add=True)
        pltpu.sync_copy(bufa_v, ss_acc.at[idxs_v.at[c]], add=True)
        cb.wait()
        pltpu.sync_copy(bufb_v, sd_acc.at[idxd_v.at[c + 1]], add=True)
        pltpu.sync_copy(bufb_v, ss_acc.at[idxs_v.at[c + 1]], add=True)
        return carry

    lax.fori_loop(0, NCH // 2, pair, 0)

    if NCH % 2:
        c = NCH - 1
        r = wid * NCH + c
        pltpu.sync_copy(xe_hbm.at[r], bufa_v)
        pltpu.sync_copy(bufa_v, sd_acc.at[idxd_v.at[c]], add=True)
        pltpu.sync_copy(bufa_v, ss_acc.at[idxs_v.at[c]], add=True)

    plsc.subcore_barrier()

    @pl.when(sid < ZT)
    def _():
        pltpu.sync_copy(sd_acc.at[zs], sdp_hbm.at[cid, zs])
        pltpu.sync_copy(ss_acc.at[zs], ssp_hbm.at[cid, zs])


def _sc_gather(p, q, dst3d, src3d):
    return pl.kernel(
        _sc_gather_body,
        out_type=jax.ShapeDtypeStruct((E // CH, CH, C), jnp.float32),
        mesh=plsc.VectorSubcoreMesh(core_axis_name="c", subcore_axis_name="s",
                                    num_cores=SC_NC, num_subcores=SC_NS),
        compiler_params=pltpu.CompilerParams(use_tc_tiling_on_sc=False),
        scratch_types=[
            pltpu.VMEM((NCH, CH), jnp.int32),
            pltpu.VMEM((NCH, CH), jnp.int32),
            pltpu.VMEM((CH, C), jnp.float32),
            pltpu.VMEM((CH, C), jnp.float32),
            pltpu.SemaphoreType.DMA,
            pltpu.SemaphoreType.DMA,
        ],
    )(p, q, dst3d, src3d)


def _sc_scatter(xe3, dst3d, src3d, zero):
    return pl.kernel(
        _sc_scatter_body,
        out_type=[jax.ShapeDtypeStruct((SC_NC, N, C), jnp.float32),
                  jax.ShapeDtypeStruct((SC_NC, N, C), jnp.float32)],
        mesh=plsc.VectorSubcoreMesh(core_axis_name="c", subcore_axis_name="s",
                                    num_cores=SC_NC, num_subcores=SC_NS),
        compiler_params=pltpu.CompilerParams(use_tc_tiling_on_sc=False),
        scratch_types=[
            pltpu.VMEM((NCH, CH), jnp.int32),
            pltpu.VMEM((NCH, CH), jnp.int32),
            pltpu.VMEM((CH, C), jnp.float32),
            pltpu.VMEM((CH, C), jnp.float32),
            pltpu.VMEM_SHARED((N, C), jnp.float32),
            pltpu.VMEM_SHARED((N, C), jnp.float32),
            pltpu.SemaphoreType.DMA,
            pltpu.SemaphoreType.DMA,
        ],
    )(xe3, dst3d, src3d, zero)


def _graph_ops(p, q, xe128, dst3d, src3d, zero):
    g3 = _sc_gather(p, q, dst3d, src3d)
    sdp, ssp = _sc_scatter(xe128.reshape(E // CH, CH, C), dst3d, src3d, zero)
    return g3.reshape(E4, 128), sdp, ssp


def kernel(xn, xe, edge_index, K1Nopen, K2Nopen, K1Eopen, K2Eopen, KNclose,
           KE1, KE2, KN1, KN2):
    xn_r = xn[0].T
    xe64 = xe[0].T.reshape(E4, 64)
    src3d = edge_index[0].reshape(SC_NW, NCH, CH)
    dst3d = edge_index[1].reshape(SC_NW, NCH, CH)
    zero = jnp.zeros((N, C), jnp.float32)

    i4 = jnp.eye(4, dtype=jnp.float32)
    A, B, Cm = KE1[:, :, 0:C], KE1[:, :, C:2 * C], KE1[:, :, 2 * C:3 * C]
    WP = jnp.transpose(0.5 * A + Cm, (0, 2, 1))
    WQ = jnp.transpose(0.5 * A - Cm, (0, 2, 1))
    WBbd = jnp.stack([jnp.kron(i4, jnp.transpose(B[i])) for i in range(NLAYER)])
    KE2bd = jnp.stack([jnp.kron(i4, jnp.transpose(KE2[i]))
                       for i in range(NLAYER)])
    D, F, Gm = KN1[:, :, 0:C], KN1[:, :, C:2 * C], KN1[:, :, 2 * C:3 * C]
    WSD = jnp.transpose(0.5 * D + F, (0, 2, 1))
    WSS = jnp.transpose(0.5 * D - F, (0, 2, 1))
    WX = jnp.transpose(Gm, (0, 2, 1))
    KN2T = jnp.transpose(KN2, (0, 2, 1))
    KNcloseT = KNclose.T

    xn_r, p, q = _node_open(xn_r, K1Nopen.T, K2Nopen.T, WP[0], WQ[0])
    xe128 = _edge_open(xe64, jnp.kron(i4, K1Eopen.T), jnp.kron(i4, K2Eopen.T))

    for i in range(NLAYER):
        g, sdp, ssp = _graph_ops(p, q, xe128, dst3d, src3d, zero)
        xe128 = _edge_path(g, xe128, WBbd[i], KE2bd[i])
        if i + 1 < NLAYER:
            wp_n, wq_n = WP[i + 1], WQ[i + 1]
        else:
            wp_n, wq_n = KNcloseT, KNcloseT
        xn_r, p, q = _node_layer(sdp, ssp, xn_r, WSD[i], WSS[i], WX[i],
                                 KN2T[i], wp_n, wq_n)

    return (p.T[None], xe128.reshape(E, C).T[None])


# speedup vs baseline: 9.9680x; 1.1937x over previous
import functools

import jax
import jax.numpy as jnp
from jax import lax
from jax.experimental import pallas as pl
from jax.experimental.pallas import tpu as pltpu
from jax.experimental.pallas import tpu_sc as plsc

N = 10000
E = 320000
C = 32
NLAYER = 4
H = 0.1
EPS = 1e-5
E4 = E // 4
TILE_E = 8000
GRID_E = E4 // TILE_E


def _stats_from(st_ref, tot):
    m = st_ref[0] / tot
    v = st_ref[1] / tot - m * m
    return m, jax.lax.rsqrt(v + EPS)


def _node_open_body(xn_ref, k1t_ref, k2t_ref, wp_ref, wq_ref,
                    xn0_ref, p_ref, q_ref):
    t = jnp.dot(xn_ref[...], k1t_ref[...], preferred_element_type=jnp.float32)
    m = jnp.mean(t)
    v = jnp.mean((t - m) ** 2)
    z = jnp.maximum((t - m) * jax.lax.rsqrt(v + EPS), 0.0)
    xn0 = jnp.dot(z, k2t_ref[...], preferred_element_type=jnp.float32)
    xn0_ref[...] = xn0
    p_ref[...] = jnp.dot(xn0, wp_ref[...], preferred_element_type=jnp.float32)
    q_ref[...] = jnp.dot(xn0, wq_ref[...], preferred_element_type=jnp.float32)


def _node_layer_body(sd_ref, ss_ref, xn_ref, wsd_ref, wss_ref, wx_ref,
                     kn2t_ref, wp_ref, wq_ref, xn1_ref, p_ref, q_ref):
    sd = sd_ref[0] + sd_ref[1]
    ss = ss_ref[0] + ss_ref[1]
    yn = (jnp.dot(sd, wsd_ref[...], preferred_element_type=jnp.float32)
          + jnp.dot(ss, wss_ref[...], preferred_element_type=jnp.float32)
          + jnp.dot(xn_ref[...], wx_ref[...], preferred_element_type=jnp.float32))
    m = jnp.mean(yn)
    v = jnp.mean((yn - m) ** 2)
    z = jnp.maximum((yn - m) * jax.lax.rsqrt(v + EPS), 0.0)
    xn1 = xn_ref[...] + H * jnp.dot(z, kn2t_ref[...],
                                    preferred_element_type=jnp.float32)
    xn1_ref[...] = xn1
    p_ref[...] = jnp.dot(xn1, wp_ref[...], preferred_element_type=jnp.float32)
    q_ref[...] = jnp.dot(xn1, wq_ref[...], preferred_element_type=jnp.float32)


def _edge_open1_body(xe_ref, w_ref, t_ref, st_ref):
    i = pl.program_id(0)
    t = jnp.dot(xe_ref[...], w_ref[...], preferred_element_type=jnp.float32)
    t_ref[...] = t

    @pl.when(i == 0)
    def _():
        st_ref[0] = 0.0
        st_ref[1] = 0.0

    st_ref[0] += jnp.sum(t)
    st_ref[1] += jnp.sum(t * t)


def _edge_open2_body(t_ref, st_ref, k2t_ref, xe_ref):
    m, rs = _stats_from(st_ref, float(C * E))
    z = jnp.maximum((t_ref[...] - m) * rs, 0.0)
    xe_ref[...] = jnp.dot(z, k2t_ref[...], preferred_element_type=jnp.float32)


def _edge_k1_body(g_ref, xe_ref, wb_ref, y1_ref, st_ref):
    i = pl.program_id(0)
    y1 = g_ref[...] + jnp.dot(xe_ref[...], wb_ref[...],
                              preferred_element_type=jnp.float32)
    y1_ref[...] = y1

    @pl.when(i == 0)
    def _():
        st_ref[0] = 0.0
        st_ref[1] = 0.0

    st_ref[0] += jnp.sum(y1)
    st_ref[1] += jnp.sum(y1 * y1)


def _edge_k2_body(y1_ref, st1_ref, ke2t_ref, st3_ref):
    i = pl.program_id(0)
    m, rs = _stats_from(st1_ref, float(C * E))
    z = jnp.maximum((y1_ref[...] - m) * rs, 0.0)
    y3 = jnp.dot(z, ke2t_ref[...], preferred_element_type=jnp.float32)

    @pl.when(i == 0)
    def _():
        st3_ref[0] = 0.0
        st3_ref[1] = 0.0

    st3_ref[0] += jnp.sum(y3)
    st3_ref[1] += jnp.sum(y3 * y3)


def _edge_k3_body(y1_ref, xe_ref, st1_ref, st3_ref, ke2t_ref, xe1_ref):
    m1, rs1 = _stats_from(st1_ref, float(C * E))
    z = jnp.maximum((y1_ref[...] - m1) * rs1, 0.0)
    y3 = jnp.dot(z, ke2t_ref[...], preferred_element_type=jnp.float32)
    m3, rs3 = _stats_from(st3_ref, float(C * E))
    xe1_ref[...] = xe_ref[...] + H * ((y3 - m3) * rs3)


def _full(shape):
    return pl.BlockSpec(shape, lambda i: tuple(0 for _ in shape))


def _node_open(xn_r, k1t, k2t, wp, wq):
    return pl.pallas_call(
        _node_open_body,
        out_shape=[jax.ShapeDtypeStruct((N, C), jnp.float32)] * 3,
    )(xn_r, k1t, k2t, wp, wq)


def _node_layer(sd, ss, xn_r, wsd, wss, wx, kn2t, wp, wq):
    return pl.pallas_call(
        _node_layer_body,
        out_shape=[jax.ShapeDtypeStruct((N, C), jnp.float32)] * 3,
    )(sd, ss, xn_r, wsd, wss, wx, kn2t, wp, wq)


def _edge_open(xe64, k1bd, k2bd):
    t, st = pl.pallas_call(
        _edge_open1_body,
        grid=(GRID_E,),
        in_specs=[pl.BlockSpec((TILE_E, 64), lambda i: (i, 0)),
                  _full((64, 128))],
        out_specs=[pl.BlockSpec((TILE_E, 128), lambda i: (i, 0)),
                   pl.BlockSpec(memory_space=pltpu.SMEM)],
        out_shape=[jax.ShapeDtypeStruct((E4, 128), jnp.float32),
                   jax.ShapeDtypeStruct((2,), jnp.float32)],
    )(xe64, k1bd)
    return pl.pallas_call(
        _edge_open2_body,
        grid=(GRID_E,),
        in_specs=[pl.BlockSpec((TILE_E, 128), lambda i: (i, 0)),
                  pl.BlockSpec(memory_space=pltpu.SMEM),
                  _full((128, 128))],
        out_specs=pl.BlockSpec((TILE_E, 128), lambda i: (i, 0)),
        out_shape=jax.ShapeDtypeStruct((E4, 128), jnp.float32),
    )(t, st, k2bd)


def _edge_path(g, xe128, wbbd, ke2bd):
    y1, st1 = pl.pallas_call(
        _edge_k1_body,
        grid=(GRID_E,),
        in_specs=[pl.BlockSpec((TILE_E, 128), lambda i: (i, 0)),
                  pl.BlockSpec((TILE_E, 128), lambda i: (i, 0)),
                  _full((128, 128))],
        out_specs=[pl.BlockSpec((TILE_E, 128), lambda i: (i, 0)),
                   pl.BlockSpec(memory_space=pltpu.SMEM)],
        out_shape=[jax.ShapeDtypeStruct((E4, 128), jnp.float32),
                   jax.ShapeDtypeStruct((2,), jnp.float32)],
    )(g, xe128, wbbd)
    st3 = pl.pallas_call(
        _edge_k2_body,
        grid=(GRID_E,),
        in_specs=[pl.BlockSpec((TILE_E, 128), lambda i: (i, 0)),
                  pl.BlockSpec(memory_space=pltpu.SMEM),
                  _full((128, 128))],
        out_specs=pl.BlockSpec(memory_space=pltpu.SMEM),
        out_shape=jax.ShapeDtypeStruct((2,), jnp.float32),
    )(y1, st1, ke2bd)
    xe1 = pl.pallas_call(
        _edge_k3_body,
        grid=(GRID_E,),
        in_specs=[pl.BlockSpec((TILE_E, 128), lambda i: (i, 0)),
                  pl.BlockSpec((TILE_E, 128), lambda i: (i, 0)),
                  pl.BlockSpec(memory_space=pltpu.SMEM),
                  pl.BlockSpec(memory_space=pltpu.SMEM),
                  _full((128, 128))],
        out_specs=pl.BlockSpec((TILE_E, 128), lambda i: (i, 0)),
        out_shape=jax.ShapeDtypeStruct((E4, 128), jnp.float32),
    )(y1, xe128, st1, st3, ke2bd)
    return xe1


SC_NC = 2
SC_NS = 16
SC_NW = SC_NC * SC_NS
EW = E // SC_NW
CH = 80
NCH = EW // CH
ZT = 10
ZROW = N // ZT


NSLOT = 4


def _sc_gather_body(p_hbm, q_hbm, dst_hbm, src_hbm, g_hbm,
                    idxd_v, idxs_v, b0, b1, b2, b3, s0, s1, s2, s3):
    cid = lax.axis_index("c")
    sid = lax.axis_index("s")
    wid = sid * SC_NC + cid

    pltpu.sync_copy(dst_hbm.at[wid], idxd_v)
    pltpu.sync_copy(src_hbm.at[wid], idxs_v)

    bufs = [b0, b1, b2, b3]
    sems = [s0, s1, s2, s3]

    def blk(t, carry):
        c0 = NSLOT * t
        r0 = wid * NCH + c0
        ds = [pltpu.async_copy(p_hbm.at[idxd_v.at[c0 + k]], bufs[k], sems[k])
              for k in range(NSLOT)]
        for k in range(NSLOT):
            ds[k].wait()
            ds[k] = pltpu.async_copy(q_hbm.at[idxs_v.at[c0 + k]], bufs[k],
                                     sems[k], add=True)
        for k in range(NSLOT):
            ds[k].wait()
            ds[k] = pltpu.async_copy(bufs[k], g_hbm.at[r0 + k], sems[k])
        for k in range(NSLOT):
            ds[k].wait()
        return carry

    lax.fori_loop(0, NCH // NSLOT, blk, 0)

    for c in range(NSLOT * (NCH // NSLOT), NCH):
        r = wid * NCH + c
        pltpu.async_copy(p_hbm.at[idxd_v.at[c]], b0, s0).wait()
        pltpu.async_copy(q_hbm.at[idxs_v.at[c]], b0, s0, add=True).wait()
        pltpu.sync_copy(b0, g_hbm.at[r])


def _sc_scatter_body(xe_hbm, dst_hbm, src_hbm, zero_hbm,
                     sdp_hbm, ssp_hbm,
                     idxd_v, idxs_v, b0, b1, b2, b3,
                     sd_acc, ss_acc, s0, s1, s2, s3, t0, t1, t2, t3):
    cid = lax.axis_index("c")
    sid = lax.axis_index("s")
    wid = sid * SC_NC + cid

    pltpu.sync_copy(dst_hbm.at[wid], idxd_v)
    pltpu.sync_copy(src_hbm.at[wid], idxs_v)

    zs = pl.ds(sid * ZROW, ZROW)

    @pl.when(sid < ZT)
    def _():
        pltpu.sync_copy(zero_hbm.at[zs], sd_acc.at[zs])
        pltpu.sync_copy(zero_hbm.at[zs], ss_acc.at[zs])

    plsc.subcore_barrier()

    bufs = [b0, b1, b2, b3]
    sems = [s0, s1, s2, s3]
    tems = [t0, t1, t2, t3]

    def blk(t, carry):
        c0 = NSLOT * t
        r0 = wid * NCH + c0
        ld = [pltpu.async_copy(xe_hbm.at[r0 + k], bufs[k], sems[k])
              for k in range(NSLOT)]
        sc = []
        for k in range(NSLOT):
            ld[k].wait()
            sc.append(pltpu.async_copy(bufs[k], sd_acc.at[idxd_v.at[c0 + k]],
                                       sems[k], add=True))
            sc.append(pltpu.async_copy(bufs[k], ss_acc.at[idxs_v.at[c0 + k]],
                                       tems[k], add=True))
        for d in sc:
            d.wait()
        return carry

    lax.fori_loop(0, NCH // NSLOT, blk, 0)

    for c in range(NSLOT * (NCH // NSLOT), NCH):
        r = wid * NCH + c
        pltpu.sync_copy(xe_hbm.at[r], b0)
        pltpu.sync_copy(b0, sd_acc.at[idxd_v.at[c]], add=True)
        pltpu.sync_copy(b0, ss_acc.at[idxs_v.at[c]], add=True)

    plsc.subcore_barrier()

    @pl.when(sid < ZT)
    def _():
        pltpu.sync_copy(sd_acc.at[zs], sdp_hbm.at[cid, zs])
        pltpu.sync_copy(ss_acc.at[zs], ssp_hbm.at[cid, zs])


def _sc_gather(p, q, dst3d, src3d):
    return pl.kernel(
        _sc_gather_body,
        out_type=jax.ShapeDtypeStruct((E // CH, CH, C), jnp.float32),
        mesh=plsc.VectorSubcoreMesh(core_axis_name="c", subcore_axis_name="s",
                                    num_cores=SC_NC, num_subcores=SC_NS),
        compiler_params=pltpu.CompilerParams(use_tc_tiling_on_sc=False),
        scratch_types=(
            [pltpu.VMEM((NCH, CH), jnp.int32)] * 2
            + [pltpu.VMEM((CH, C), jnp.float32)] * NSLOT
            + [pltpu.SemaphoreType.DMA] * NSLOT
        ),
    )(p, q, dst3d, src3d)


def _sc_scatter(xe3, dst3d, src3d, zero):
    return pl.kernel(
        _sc_scatter_body,
        out_type=[jax.ShapeDtypeStruct((SC_NC, N, C), jnp.float32),
                  jax.ShapeDtypeStruct((SC_NC, N, C), jnp.float32)],
        mesh=plsc.VectorSubcoreMesh(core_axis_name="c", subcore_axis_name="s",
                                    num_cores=SC_NC, num_subcores=SC_NS),
        compiler_params=pltpu.CompilerParams(use_tc_tiling_on_sc=False),
        scratch_types=(
            [pltpu.VMEM((NCH, CH), jnp.int32)] * 2
            + [pltpu.VMEM((CH, C), jnp.float32)] * NSLOT
            + [pltpu.VMEM_SHARED((N, C), jnp.float32)] * 2
            + [pltpu.SemaphoreType.DMA] * (2 * NSLOT)
        ),
    )(xe3, dst3d, src3d, zero)


def _graph_ops(p, q, xe128, dst3d, src3d, zero):
    g3 = _sc_gather(p, q, dst3d, src3d)
    sdp, ssp = _sc_scatter(xe128.reshape(E // CH, CH, C), dst3d, src3d, zero)
    return g3.reshape(E4, 128), sdp, ssp


def kernel(xn, xe, edge_index, K1Nopen, K2Nopen, K1Eopen, K2Eopen, KNclose,
           KE1, KE2, KN1, KN2):
    xn_r = xn[0].T
    xe64 = xe[0].T.reshape(E4, 64)
    src3d = edge_index[0].reshape(SC_NW, NCH, CH)
    dst3d = edge_index[1].reshape(SC_NW, NCH, CH)
    zero = jnp.zeros((N, C), jnp.float32)

    i4 = jnp.eye(4, dtype=jnp.float32)
    A, B, Cm = KE1[:, :, 0:C], KE1[:, :, C:2 * C], KE1[:, :, 2 * C:3 * C]
    WP = jnp.transpose(0.5 * A + Cm, (0, 2, 1))
    WQ = jnp.transpose(0.5 * A - Cm, (0, 2, 1))
    WBbd = jnp.stack([jnp.kron(i4, jnp.transpose(B[i])) for i in range(NLAYER)])
    KE2bd = jnp.stack([jnp.kron(i4, jnp.transpose(KE2[i]))
                       for i in range(NLAYER)])
    D, F, Gm = KN1[:, :, 0:C], KN1[:, :, C:2 * C], KN1[:, :, 2 * C:3 * C]
    WSD = jnp.transpose(0.5 * D + F, (0, 2, 1))
    WSS = jnp.transpose(0.5 * D - F, (0, 2, 1))
    WX = jnp.transpose(Gm, (0, 2, 1))
    KN2T = jnp.transpose(KN2, (0, 2, 1))
    KNcloseT = KNclose.T

    xn_r, p, q = _node_open(xn_r, K1Nopen.T, K2Nopen.T, WP[0], WQ[0])
    xe128 = _edge_open(xe64, jnp.kron(i4, K1Eopen.T), jnp.kron(i4, K2Eopen.T))

    for i in range(NLAYER):
        g, sdp, ssp = _graph_ops(p, q, xe128, dst3d, src3d, zero)
        xe128 = _edge_path(g, xe128, WBbd[i], KE2bd[i])
        if i + 1 < NLAYER:
            wp_n, wq_n = WP[i + 1], WQ[i + 1]
        else:
            wp_n, wq_n = KNcloseT, KNcloseT
        xn_r, p, q = _node_layer(sdp, ssp, xn_r, WSD[i], WSS[i], WX[i],
                                 KN2T[i], wp_n, wq_n)

    return (p.T[None], xe128.reshape(E, C).T[None])


# speedup vs baseline: 11.1618x; 1.1198x over previous
import functools

import jax
import jax.numpy as jnp
from jax import lax
from jax.experimental import pallas as pl
from jax.experimental.pallas import tpu as pltpu
from jax.experimental.pallas import tpu_sc as plsc

N = 10000
E = 320000
C = 32
NLAYER = 4
H = 0.1
EPS = 1e-5
E4 = E // 4
TILE_E = 8000
GRID_E = E4 // TILE_E


def _stats_from(st_ref, tot):
    m = st_ref[0] / tot
    v = st_ref[1] / tot - m * m
    return m, jax.lax.rsqrt(v + EPS)


def _node_open_body(xn_ref, k1t_ref, k2t_ref, wp_ref, wq_ref,
                    xn0_ref, p_ref, q_ref):
    t = jnp.dot(xn_ref[...], k1t_ref[...], preferred_element_type=jnp.float32)
    m = jnp.mean(t)
    v = jnp.mean((t - m) ** 2)
    z = jnp.maximum((t - m) * jax.lax.rsqrt(v + EPS), 0.0)
    xn0 = jnp.dot(z, k2t_ref[...], preferred_element_type=jnp.float32)
    xn0_ref[...] = xn0
    p_ref[...] = jnp.dot(xn0, wp_ref[...], preferred_element_type=jnp.float32)
    q_ref[...] = jnp.dot(xn0, wq_ref[...], preferred_element_type=jnp.float32)


def _node_layer_body(sd_ref, ss_ref, xn_ref, wsd_ref, wss_ref, wx_ref,
                     kn2t_ref, wp_ref, wq_ref, xn1_ref, p_ref, q_ref):
    sd = sd_ref[0] + sd_ref[1]
    ss = ss_ref[0] + ss_ref[1]
    yn = (jnp.dot(sd, wsd_ref[...], preferred_element_type=jnp.float32)
          + jnp.dot(ss, wss_ref[...], preferred_element_type=jnp.float32)
          + jnp.dot(xn_ref[...], wx_ref[...], preferred_element_type=jnp.float32))
    m = jnp.mean(yn)
    v = jnp.mean((yn - m) ** 2)
    z = jnp.maximum((yn - m) * jax.lax.rsqrt(v + EPS), 0.0)
    xn1 = xn_ref[...] + H * jnp.dot(z, kn2t_ref[...],
                                    preferred_element_type=jnp.float32)
    xn1_ref[...] = xn1
    p_ref[...] = jnp.dot(xn1, wp_ref[...], preferred_element_type=jnp.float32)
    q_ref[...] = jnp.dot(xn1, wq_ref[...], preferred_element_type=jnp.float32)


def _edge_open1_body(xe_ref, w_ref, t_ref, st_ref):
    i = pl.program_id(0)
    t = jnp.dot(xe_ref[...], w_ref[...], preferred_element_type=jnp.float32)
    t_ref[...] = t

    @pl.when(i == 0)
    def _():
        st_ref[0] = 0.0
        st_ref[1] = 0.0

    st_ref[0] += jnp.sum(t)
    st_ref[1] += jnp.sum(t * t)


def _edge_open2_body(t_ref, st_ref, k2t_ref, xe_ref):
    m, rs = _stats_from(st_ref, float(C * E))
    z = jnp.maximum((t_ref[...] - m) * rs, 0.0)
    xe_ref[...] = jnp.dot(z, k2t_ref[...], preferred_element_type=jnp.float32)


def _edge_k1_body(g_ref, xe_ref, wb_ref, y1_ref, st_ref):
    i = pl.program_id(0)
    y1 = g_ref[...] + jnp.dot(xe_ref[...], wb_ref[...],
                              preferred_element_type=jnp.float32)
    y1_ref[...] = y1

    @pl.when(i == 0)
    def _():
        st_ref[0] = 0.0
        st_ref[1] = 0.0

    st_ref[0] += jnp.sum(y1)
    st_ref[1] += jnp.sum(y1 * y1)


def _edge_k2_body(y1_ref, st1_ref, ke2t_ref, st3_ref):
    i = pl.program_id(0)
    m, rs = _stats_from(st1_ref, float(C * E))
    z = jnp.maximum((y1_ref[...] - m) * rs, 0.0)
    y3 = jnp.dot(z, ke2t_ref[...], preferred_element_type=jnp.float32)

    @pl.when(i == 0)
    def _():
        st3_ref[0] = 0.0
        st3_ref[1] = 0.0

    st3_ref[0] += jnp.sum(y3)
    st3_ref[1] += jnp.sum(y3 * y3)


def _edge_k3_body(y1_ref, xe_ref, st1_ref, st3_ref, ke2t_ref, xe1_ref):
    m1, rs1 = _stats_from(st1_ref, float(C * E))
    z = jnp.maximum((y1_ref[...] - m1) * rs1, 0.0)
    y3 = jnp.dot(z, ke2t_ref[...], preferred_element_type=jnp.float32)
    m3, rs3 = _stats_from(st3_ref, float(C * E))
    xe1_ref[...] = xe_ref[...] + H * ((y3 - m3) * rs3)


def _full(shape):
    return pl.BlockSpec(shape, lambda i: tuple(0 for _ in shape))


def _node_open(xn_r, k1t, k2t, wp, wq):
    return pl.pallas_call(
        _node_open_body,
        out_shape=[jax.ShapeDtypeStruct((N, C), jnp.float32)] * 3,
    )(xn_r, k1t, k2t, wp, wq)


def _node_layer(sd, ss, xn_r, wsd, wss, wx, kn2t, wp, wq):
    return pl.pallas_call(
        _node_layer_body,
        out_shape=[jax.ShapeDtypeStruct((N, C), jnp.float32)] * 3,
    )(sd, ss, xn_r, wsd, wss, wx, kn2t, wp, wq)


def _edge_open(xe64, k1bd, k2bd):
    t, st = pl.pallas_call(
        _edge_open1_body,
        grid=(GRID_E,),
        in_specs=[pl.BlockSpec((TILE_E, 64), lambda i: (i, 0)),
                  _full((64, 128))],
        out_specs=[pl.BlockSpec((TILE_E, 128), lambda i: (i, 0)),
                   pl.BlockSpec(memory_space=pltpu.SMEM)],
        out_shape=[jax.ShapeDtypeStruct((E4, 128), jnp.float32),
                   jax.ShapeDtypeStruct((2,), jnp.float32)],
    )(xe64, k1bd)
    return pl.pallas_call(
        _edge_open2_body,
        grid=(GRID_E,),
        in_specs=[pl.BlockSpec((TILE_E, 128), lambda i: (i, 0)),
                  pl.BlockSpec(memory_space=pltpu.SMEM),
                  _full((128, 128))],
        out_specs=pl.BlockSpec((TILE_E, 128), lambda i: (i, 0)),
        out_shape=jax.ShapeDtypeStruct((E4, 128), jnp.float32),
    )(t, st, k2bd)


def _edge_path(g, xe128, wbbd, ke2bd):
    y1, st1 = pl.pallas_call(
        _edge_k1_body,
        grid=(GRID_E,),
        in_specs=[pl.BlockSpec((TILE_E, 128), lambda i: (i, 0)),
                  pl.BlockSpec((TILE_E, 128), lambda i: (i, 0)),
                  _full((128, 128))],
        out_specs=[pl.BlockSpec((TILE_E, 128), lambda i: (i, 0)),
                   pl.BlockSpec(memory_space=pltpu.SMEM)],
        out_shape=[jax.ShapeDtypeStruct((E4, 128), jnp.float32),
                   jax.ShapeDtypeStruct((2,), jnp.float32)],
    )(g, xe128, wbbd)
    st3 = pl.pallas_call(
        _edge_k2_body,
        grid=(GRID_E,),
        in_specs=[pl.BlockSpec((TILE_E, 128), lambda i: (i, 0)),
                  pl.BlockSpec(memory_space=pltpu.SMEM),
                  _full((128, 128))],
        out_specs=pl.BlockSpec(memory_space=pltpu.SMEM),
        out_shape=jax.ShapeDtypeStruct((2,), jnp.float32),
    )(y1, st1, ke2bd)
    xe1 = pl.pallas_call(
        _edge_k3_body,
        grid=(GRID_E,),
        in_specs=[pl.BlockSpec((TILE_E, 128), lambda i: (i, 0)),
                  pl.BlockSpec((TILE_E, 128), lambda i: (i, 0)),
                  pl.BlockSpec(memory_space=pltpu.SMEM),
                  pl.BlockSpec(memory_space=pltpu.SMEM),
                  _full((128, 128))],
        out_specs=pl.BlockSpec((TILE_E, 128), lambda i: (i, 0)),
        out_shape=jax.ShapeDtypeStruct((E4, 128), jnp.float32),
    )(y1, xe128, st1, st3, ke2bd)
    return xe1


SC_NC = 2
SC_NS = 16
SC_NW = SC_NC * SC_NS
EW = E // SC_NW
CH = 100
NCH = EW // CH
ZT = 10
ZROW = N // ZT


NSLOT = 4


def _sc_gather_body(p_hbm, q_hbm, dst_hbm, src_hbm, g_hbm,
                    idxd_v, idxs_v, b0, b1, b2, b3, p_sh, q_sh,
                    s0, s1, s2, s3):
    cid = lax.axis_index("c")
    sid = lax.axis_index("s")
    wid = sid * SC_NC + cid

    pltpu.sync_copy(dst_hbm.at[wid], idxd_v)
    pltpu.sync_copy(src_hbm.at[wid], idxs_v)

    def stage(part):
        @pl.when(part < ZT)
        def _():
            rs = pl.ds(part * ZROW, ZROW)
            pltpu.sync_copy(p_hbm.at[rs], p_sh.at[rs])

        @pl.when((part >= ZT) & (part < 2 * ZT))
        def _():
            rs = pl.ds((part - ZT) * ZROW, ZROW)
            pltpu.sync_copy(q_hbm.at[rs], q_sh.at[rs])

    stage(sid)
    stage(sid + SC_NS)
    plsc.subcore_barrier()

    bufs = [b0, b1, b2, b3]
    sems = [s0, s1, s2, s3]

    def blk(t, carry):
        c0 = NSLOT * t
        r0 = wid * NCH + c0
        ds = [pltpu.async_copy(p_sh.at[idxd_v.at[c0 + k]], bufs[k], sems[k])
              for k in range(NSLOT)]
        for k in range(NSLOT):
            ds[k].wait()
            ds[k] = pltpu.async_copy(q_sh.at[idxs_v.at[c0 + k]], bufs[k],
                                     sems[k], add=True)
        for k in range(NSLOT):
            ds[k].wait()
            ds[k] = pltpu.async_copy(bufs[k], g_hbm.at[r0 + k], sems[k])
        for k in range(NSLOT):
            ds[k].wait()
        return carry

    lax.fori_loop(0, NCH // NSLOT, blk, 0)

    for c in range(NSLOT * (NCH // NSLOT), NCH):
        r = wid * NCH + c
        pltpu.async_copy(p_sh.at[idxd_v.at[c]], b0, s0).wait()
        pltpu.async_copy(q_sh.at[idxs_v.at[c]], b0, s0, add=True).wait()
        pltpu.sync_copy(b0, g_hbm.at[r])


def _sc_scatter_body(xe_hbm, dst_hbm, src_hbm, zero_hbm,
                     sdp_hbm, ssp_hbm,
                     idxd_v, idxs_v, b0, b1, b2, b3,
                     sd_acc, ss_acc, s0, s1, s2, s3, t0, t1, t2, t3):
    cid = lax.axis_index("c")
    sid = lax.axis_index("s")
    wid = sid * SC_NC + cid

    pltpu.sync_copy(dst_hbm.at[wid], idxd_v)
    pltpu.sync_copy(src_hbm.at[wid], idxs_v)

    zs = pl.ds(sid * ZROW, ZROW)

    @pl.when(sid < ZT)
    def _():
        pltpu.sync_copy(zero_hbm.at[zs], sd_acc.at[zs])
        pltpu.sync_copy(zero_hbm.at[zs], ss_acc.at[zs])

    plsc.subcore_barrier()

    bufs = [b0, b1, b2, b3]
    sems = [s0, s1, s2, s3]
    tems = [t0, t1, t2, t3]

    def blk(t, carry):
        c0 = NSLOT * t
        r0 = wid * NCH + c0
        ld = [pltpu.async_copy(xe_hbm.at[r0 + k], bufs[k], sems[k])
              for k in range(NSLOT)]
        sc = []
        for k in range(NSLOT):
            ld[k].wait()
            sc.append(pltpu.async_copy(bufs[k], sd_acc.at[idxd_v.at[c0 + k]],
                                       sems[k], add=True))
            sc.append(pltpu.async_copy(bufs[k], ss_acc.at[idxs_v.at[c0 + k]],
                                       tems[k], add=True))
        for d in sc:
            d.wait()
        return carry

    lax.fori_loop(0, NCH // NSLOT, blk, 0)

    for c in range(NSLOT * (NCH // NSLOT), NCH):
        r = wid * NCH + c
        pltpu.sync_copy(xe_hbm.at[r], b0)
        pltpu.sync_copy(b0, sd_acc.at[idxd_v.at[c]], add=True)
        pltpu.sync_copy(b0, ss_acc.at[idxs_v.at[c]], add=True)

    plsc.subcore_barrier()

    @pl.when(sid < ZT)
    def _():
        pltpu.sync_copy(sd_acc.at[zs], sdp_hbm.at[cid, zs])
        pltpu.sync_copy(ss_acc.at[zs], ssp_hbm.at[cid, zs])


def _sc_gather(p, q, dst3d, src3d):
    return pl.kernel(
        _sc_gather_body,
        out_type=jax.ShapeDtypeStruct((E // CH, CH, C), jnp.float32),
        mesh=plsc.VectorSubcoreMesh(core_axis_name="c", subcore_axis_name="s",
                                    num_cores=SC_NC, num_subcores=SC_NS),
        compiler_params=pltpu.CompilerParams(use_tc_tiling_on_sc=False),
        scratch_types=(
            [pltpu.VMEM((NCH, CH), jnp.int32)] * 2
            + [pltpu.VMEM((CH, C), jnp.float32)] * NSLOT
            + [pltpu.VMEM_SHARED((N, C), jnp.float32)] * 2
            + [pltpu.SemaphoreType.DMA] * NSLOT
        ),
    )(p, q, dst3d, src3d)


def _sc_scatter(xe3, dst3d, src3d, zero):
    return pl.kernel(
        _sc_scatter_body,
        out_type=[jax.ShapeDtypeStruct((SC_NC, N, C), jnp.float32),
                  jax.ShapeDtypeStruct((SC_NC, N, C), jnp.float32)],
        mesh=plsc.VectorSubcoreMesh(core_axis_name="c", subcore_axis_name="s",
                                    num_cores=SC_NC, num_subcores=SC_NS),
        compiler_params=pltpu.CompilerParams(use_tc_tiling_on_sc=False),
        scratch_types=(
            [pltpu.VMEM((NCH, CH), jnp.int32)] * 2
            + [pltpu.VMEM((CH, C), jnp.float32)] * NSLOT
            + [pltpu.VMEM_SHARED((N, C), jnp.float32)] * 2
            + [pltpu.SemaphoreType.DMA] * (2 * NSLOT)
        ),
    )(xe3, dst3d, src3d, zero)


def _graph_ops(p, q, xe128, dst3d, src3d, zero):
    g3 = _sc_gather(p, q, dst3d, src3d)
    sdp, ssp = _sc_scatter(xe128.reshape(E // CH, CH, C), dst3d, src3d, zero)
    return g3.reshape(E4, 128), sdp, ssp


def kernel(xn, xe, edge_index, K1Nopen, K2Nopen, K1Eopen, K2Eopen, KNclose,
           KE1, KE2, KN1, KN2):
    xn_r = xn[0].T
    xe64 = xe[0].T.reshape(E4, 64)
    src3d = edge_index[0].reshape(SC_NW, NCH, CH)
    dst3d = edge_index[1].reshape(SC_NW, NCH, CH)
    zero = jnp.zeros((N, C), jnp.float32)

    i4 = jnp.eye(4, dtype=jnp.float32)
    A, B, Cm = KE1[:, :, 0:C], KE1[:, :, C:2 * C], KE1[:, :, 2 * C:3 * C]
    WP = jnp.transpose(0.5 * A + Cm, (0, 2, 1))
    WQ = jnp.transpose(0.5 * A - Cm, (0, 2, 1))
    WBbd = jnp.stack([jnp.kron(i4, jnp.transpose(B[i])) for i in range(NLAYER)])
    KE2bd = jnp.stack([jnp.kron(i4, jnp.transpose(KE2[i]))
                       for i in range(NLAYER)])
    D, F, Gm = KN1[:, :, 0:C], KN1[:, :, C:2 * C], KN1[:, :, 2 * C:3 * C]
    WSD = jnp.transpose(0.5 * D + F, (0, 2, 1))
    WSS = jnp.transpose(0.5 * D - F, (0, 2, 1))
    WX = jnp.transpose(Gm, (0, 2, 1))
    KN2T = jnp.transpose(KN2, (0, 2, 1))
    KNcloseT = KNclose.T

    xn_r, p, q = _node_open(xn_r, K1Nopen.T, K2Nopen.T, WP[0], WQ[0])
    xe128 = _edge_open(xe64, jnp.kron(i4, K1Eopen.T), jnp.kron(i4, K2Eopen.T))

    for i in range(NLAYER):
        g, sdp, ssp = _graph_ops(p, q, xe128, dst3d, src3d, zero)
        xe128 = _edge_path(g, xe128, WBbd[i], KE2bd[i])
        if i + 1 < NLAYER:
            wp_n, wq_n = WP[i + 1], WQ[i + 1]
        else:
            wp_n, wq_n = KNcloseT, KNcloseT
        xn_r, p, q = _node_layer(sdp, ssp, xn_r, WSD[i], WSS[i], WX[i],
                                 KN2T[i], wp_n, wq_n)

    return (p.T[None], xe128.reshape(E, C).T[None])


# speedup vs baseline: 11.7121x; 1.0493x over previous
import functools

import jax
import jax.numpy as jnp
from jax import lax
from jax.experimental import pallas as pl
from jax.experimental.pallas import tpu as pltpu
from jax.experimental.pallas import tpu_sc as plsc

N = 10000
E = 320000
C = 32
NLAYER = 4
H = 0.1
EPS = 1e-5
E4 = E // 4
TILE_E = 8000
GRID_E = E4 // TILE_E


def _stats_from(st_ref, tot):
    m = st_ref[0] / tot
    v = st_ref[1] / tot - m * m
    return m, jax.lax.rsqrt(v + EPS)


def _node_open_body(xn_ref, k1t_ref, k2t_ref, wp_ref, wq_ref,
                    xn0_ref, p_ref, q_ref):
    t = jnp.dot(xn_ref[...], k1t_ref[...], preferred_element_type=jnp.float32)
    m = jnp.mean(t)
    v = jnp.mean((t - m) ** 2)
    z = jnp.maximum((t - m) * jax.lax.rsqrt(v + EPS), 0.0)
    xn0 = jnp.dot(z, k2t_ref[...], preferred_element_type=jnp.float32)
    xn0_ref[...] = xn0
    p_ref[...] = jnp.dot(xn0, wp_ref[...], preferred_element_type=jnp.float32)
    q_ref[...] = jnp.dot(xn0, wq_ref[...], preferred_element_type=jnp.float32)


def _node_layer_body(sd_ref, ss_ref, xn_ref, wsd_ref, wss_ref, wx_ref,
                     kn2t_ref, wp_ref, wq_ref, xn1_ref, p_ref, q_ref):
    sd = sd_ref[0] + sd_ref[1]
    ss = ss_ref[0] + ss_ref[1]
    yn = (jnp.dot(sd, wsd_ref[...], preferred_element_type=jnp.float32)
          + jnp.dot(ss, wss_ref[...], preferred_element_type=jnp.float32)
          + jnp.dot(xn_ref[...], wx_ref[...], preferred_element_type=jnp.float32))
    m = jnp.mean(yn)
    v = jnp.mean((yn - m) ** 2)
    z = jnp.maximum((yn - m) * jax.lax.rsqrt(v + EPS), 0.0)
    xn1 = xn_ref[...] + H * jnp.dot(z, kn2t_ref[...],
                                    preferred_element_type=jnp.float32)
    xn1_ref[...] = xn1
    p_ref[...] = jnp.dot(xn1, wp_ref[...], preferred_element_type=jnp.float32)
    q_ref[...] = jnp.dot(xn1, wq_ref[...], preferred_element_type=jnp.float32)


def _edge_open1_body(xe_ref, w_ref, t_ref, st_ref):
    i = pl.program_id(0)
    t = jnp.dot(xe_ref[...], w_ref[...], preferred_element_type=jnp.float32)
    t_ref[...] = t

    @pl.when(i == 0)
    def _():
        st_ref[0] = 0.0
        st_ref[1] = 0.0

    st_ref[0] += jnp.sum(t)
    st_ref[1] += jnp.sum(t * t)


def _edge_open2_body(t_ref, st_ref, k2t_ref, xe_ref):
    m, rs = _stats_from(st_ref, float(C * E))
    z = jnp.maximum((t_ref[...] - m) * rs, 0.0)
    xe_ref[...] = jnp.dot(z, k2t_ref[...], preferred_element_type=jnp.float32)


def _edge_k1_body(g_ref, xe_ref, wb_ref, y1_ref, st_ref):
    i = pl.program_id(0)
    y1 = g_ref[...] + jnp.dot(xe_ref[...], wb_ref[...],
                              preferred_element_type=jnp.float32)
    y1_ref[...] = y1.astype(jnp.bfloat16)

    @pl.when(i == 0)
    def _():
        st_ref[0] = 0.0
        st_ref[1] = 0.0

    st_ref[0] += jnp.sum(y1)
    st_ref[1] += jnp.sum(y1 * y1)


def _edge_k2_body(y1_ref, st1_ref, ke2t_ref, st3_ref):
    i = pl.program_id(0)
    m, rs = _stats_from(st1_ref, float(C * E))
    z = jnp.maximum((y1_ref[...].astype(jnp.float32) - m) * rs, 0.0)
    y3 = jnp.dot(z, ke2t_ref[...], preferred_element_type=jnp.float32)

    @pl.when(i == 0)
    def _():
        st3_ref[0] = 0.0
        st3_ref[1] = 0.0

    st3_ref[0] += jnp.sum(y3)
    st3_ref[1] += jnp.sum(y3 * y3)


def _edge_k3_body(y1_ref, xe_ref, st1_ref, st3_ref, ke2t_ref, xe1_ref):
    m1, rs1 = _stats_from(st1_ref, float(C * E))
    z = jnp.maximum((y1_ref[...].astype(jnp.float32) - m1) * rs1, 0.0)
    y3 = jnp.dot(z, ke2t_ref[...], preferred_element_type=jnp.float32)
    m3, rs3 = _stats_from(st3_ref, float(C * E))
    xe1_ref[...] = xe_ref[...] + H * ((y3 - m3) * rs3)


def _full(shape):
    return pl.BlockSpec(shape, lambda i: tuple(0 for _ in shape))


def _node_open(xn_r, k1t, k2t, wp, wq):
    return pl.pallas_call(
        _node_open_body,
        out_shape=[jax.ShapeDtypeStruct((N, C), jnp.float32)] * 3,
    )(xn_r, k1t, k2t, wp, wq)


def _node_layer(sd, ss, xn_r, wsd, wss, wx, kn2t, wp, wq):
    return pl.pallas_call(
        _node_layer_body,
        out_shape=[jax.ShapeDtypeStruct((N, C), jnp.float32)] * 3,
    )(sd, ss, xn_r, wsd, wss, wx, kn2t, wp, wq)


def _edge_open(xe64, k1bd, k2bd):
    t, st = pl.pallas_call(
        _edge_open1_body,
        grid=(GRID_E,),
        in_specs=[pl.BlockSpec((TILE_E, 64), lambda i: (i, 0)),
                  _full((64, 128))],
        out_specs=[pl.BlockSpec((TILE_E, 128), lambda i: (i, 0)),
                   pl.BlockSpec(memory_space=pltpu.SMEM)],
        out_shape=[jax.ShapeDtypeStruct((E4, 128), jnp.float32),
                   jax.ShapeDtypeStruct((2,), jnp.float32)],
    )(xe64, k1bd)
    return pl.pallas_call(
        _edge_open2_body,
        grid=(GRID_E,),
        in_specs=[pl.BlockSpec((TILE_E, 128), lambda i: (i, 0)),
                  pl.BlockSpec(memory_space=pltpu.SMEM),
                  _full((128, 128))],
        out_specs=pl.BlockSpec((TILE_E, 128), lambda i: (i, 0)),
        out_shape=jax.ShapeDtypeStruct((E4, 128), jnp.float32),
    )(t, st, k2bd)


def _edge_path(g, xe128, wbbd, ke2bd):
    y1, st1 = pl.pallas_call(
        _edge_k1_body,
        grid=(GRID_E,),
        in_specs=[pl.BlockSpec((TILE_E, 128), lambda i: (i, 0)),
                  pl.BlockSpec((TILE_E, 128), lambda i: (i, 0)),
                  _full((128, 128))],
        out_specs=[pl.BlockSpec((TILE_E, 128), lambda i: (i, 0)),
                   pl.BlockSpec(memory_space=pltpu.SMEM)],
        out_shape=[jax.ShapeDtypeStruct((E4, 128), jnp.bfloat16),
                   jax.ShapeDtypeStruct((2,), jnp.float32)],
    )(g, xe128, wbbd)
    st3 = pl.pallas_call(
        _edge_k2_body,
        grid=(GRID_E,),
        in_specs=[pl.BlockSpec((TILE_E, 128), lambda i: (i, 0)),
                  pl.BlockSpec(memory_space=pltpu.SMEM),
                  _full((128, 128))],
        out_specs=pl.BlockSpec(memory_space=pltpu.SMEM),
        out_shape=jax.ShapeDtypeStruct((2,), jnp.float32),
    )(y1, st1, ke2bd)
    xe1 = pl.pallas_call(
        _edge_k3_body,
        grid=(GRID_E,),
        in_specs=[pl.BlockSpec((TILE_E, 128), lambda i: (i, 0)),
                  pl.BlockSpec((TILE_E, 128), lambda i: (i, 0)),
                  pl.BlockSpec(memory_space=pltpu.SMEM),
                  pl.BlockSpec(memory_space=pltpu.SMEM),
                  _full((128, 128))],
        out_specs=pl.BlockSpec((TILE_E, 128), lambda i: (i, 0)),
        out_shape=jax.ShapeDtypeStruct((E4, 128), jnp.float32),
    )(y1, xe128, st1, st3, ke2bd)
    return xe1


SC_NC = 2
SC_NS = 16
SC_NW = SC_NC * SC_NS
EW = E // SC_NW
CH = 100
NCH = EW // CH
ZT = 10
ZROW = N // ZT


NSLOT = 4


def _sc_gather_body(p_hbm, q_hbm, dst_hbm, src_hbm, g_hbm,
                    idxd_v, idxs_v, b0, b1, b2, b3, p_sh, q_sh,
                    s0, s1, s2, s3):
    cid = lax.axis_index("c")
    sid = lax.axis_index("s")
    wid = sid * SC_NC + cid

    pltpu.sync_copy(dst_hbm.at[wid], idxd_v)
    pltpu.sync_copy(src_hbm.at[wid], idxs_v)

    def stage(part):
        @pl.when(part < ZT)
        def _():
            rs = pl.ds(part * ZROW, ZROW)
            pltpu.sync_copy(p_hbm.at[rs], p_sh.at[rs])

        @pl.when((part >= ZT) & (part < 2 * ZT))
        def _():
            rs = pl.ds((part - ZT) * ZROW, ZROW)
            pltpu.sync_copy(q_hbm.at[rs], q_sh.at[rs])

    stage(sid)
    stage(sid + SC_NS)
    plsc.subcore_barrier()

    bufs = [b0, b1, b2, b3]
    sems = [s0, s1, s2, s3]

    def blk(t, carry):
        c0 = NSLOT * t
        r0 = wid * NCH + c0
        ds = [pltpu.async_copy(p_sh.at[idxd_v.at[c0 + k]], bufs[k], sems[k])
              for k in range(NSLOT)]
        for k in range(NSLOT):
            ds[k].wait()
            ds[k] = pltpu.async_copy(q_sh.at[idxs_v.at[c0 + k]], bufs[k],
                                     sems[k], add=True)
        for k in range(NSLOT):
            ds[k].wait()
            ds[k] = pltpu.async_copy(bufs[k], g_hbm.at[r0 + k], sems[k])
        for k in range(NSLOT):
            ds[k].wait()
        return carry

    lax.fori_loop(0, NCH // NSLOT, blk, 0)

    for c in range(NSLOT * (NCH // NSLOT), NCH):
        r = wid * NCH + c
        pltpu.async_copy(p_sh.at[idxd_v.at[c]], b0, s0).wait()
        pltpu.async_copy(q_sh.at[idxs_v.at[c]], b0, s0, add=True).wait()
        pltpu.sync_copy(b0, g_hbm.at[r])


def _sc_scatter_body(xe_hbm, dst_hbm, src_hbm, zero_hbm,
                     sdp_hbm, ssp_hbm,
                     idxd_v, idxs_v, b0, b1, b2, b3,
                     sd_acc, ss_acc, s0, s1, s2, s3, t0, t1, t2, t3):
    cid = lax.axis_index("c")
    sid = lax.axis_index("s")
    wid = sid * SC_NC + cid

    pltpu.sync_copy(dst_hbm.at[wid], idxd_v)
    pltpu.sync_copy(src_hbm.at[wid], idxs_v)

    zs = pl.ds(sid * ZROW, ZROW)

    @pl.when(sid < ZT)
    def _():
        pltpu.sync_copy(zero_hbm.at[zs], sd_acc.at[zs])
        pltpu.sync_copy(zero_hbm.at[zs], ss_acc.at[zs])

    plsc.subcore_barrier()

    bufs = [b0, b1, b2, b3]
    sems = [s0, s1, s2, s3]
    tems = [t0, t1, t2, t3]

    def blk(t, carry):
        c0 = NSLOT * t
        r0 = wid * NCH + c0
        ld = [pltpu.async_copy(xe_hbm.at[r0 + k], bufs[k], sems[k])
              for k in range(NSLOT)]
        sc = []
        for k in range(NSLOT):
            ld[k].wait()
            sc.append(pltpu.async_copy(bufs[k], sd_acc.at[idxd_v.at[c0 + k]],
                                       sems[k], add=True))
            sc.append(pltpu.async_copy(bufs[k], ss_acc.at[idxs_v.at[c0 + k]],
                                       tems[k], add=True))
        for d in sc:
            d.wait()
        return carry

    lax.fori_loop(0, NCH // NSLOT, blk, 0)

    for c in range(NSLOT * (NCH // NSLOT), NCH):
        r = wid * NCH + c
        pltpu.sync_copy(xe_hbm.at[r], b0)
        pltpu.sync_copy(b0, sd_acc.at[idxd_v.at[c]], add=True)
        pltpu.sync_copy(b0, ss_acc.at[idxs_v.at[c]], add=True)

    plsc.subcore_barrier()

    @pl.when(sid < ZT)
    def _():
        pltpu.sync_copy(sd_acc.at[zs], sdp_hbm.at[cid, zs])
        pltpu.sync_copy(ss_acc.at[zs], ssp_hbm.at[cid, zs])


def _sc_gather(p, q, dst3d, src3d):
    return pl.kernel(
        _sc_gather_body,
        out_type=jax.ShapeDtypeStruct((E // CH, CH, C), jnp.float32),
        mesh=plsc.VectorSubcoreMesh(core_axis_name="c", subcore_axis_name="s",
                                    num_cores=SC_NC, num_subcores=SC_NS),
        compiler_params=pltpu.CompilerParams(use_tc_tiling_on_sc=False),
        scratch_types=(
            [pltpu.VMEM((NCH, CH), jnp.int32)] * 2
            + [pltpu.VMEM((CH, C), jnp.float32)] * NSLOT
            + [pltpu.VMEM_SHARED((N, C), jnp.float32)] * 2
            + [pltpu.SemaphoreType.DMA] * NSLOT
        ),
    )(p, q, dst3d, src3d)


def _sc_scatter(xe3, dst3d, src3d, zero):
    return pl.kernel(
        _sc_scatter_body,
        out_type=[jax.ShapeDtypeStruct((SC_NC, N, C), jnp.float32),
                  jax.ShapeDtypeStruct((SC_NC, N, C), jnp.float32)],
        mesh=plsc.VectorSubcoreMesh(core_axis_name="c", subcore_axis_name="s",
                                    num_cores=SC_NC, num_subcores=SC_NS),
        compiler_params=pltpu.CompilerParams(use_tc_tiling_on_sc=False),
        scratch_types=(
            [pltpu.VMEM((NCH, CH), jnp.int32)] * 2
            + [pltpu.VMEM((CH, C), jnp.float32)] * NSLOT
            + [pltpu.VMEM_SHARED((N, C), jnp.float32)] * 2
            + [pltpu.SemaphoreType.DMA] * (2 * NSLOT)
        ),
    )(xe3, dst3d, src3d, zero)


def _graph_ops(p, q, xe128, dst3d, src3d, zero):
    g3 = _sc_gather(p, q, dst3d, src3d)
    sdp, ssp = _sc_scatter(xe128.reshape(E // CH, CH, C), dst3d, src3d, zero)
    return g3.reshape(E4, 128), sdp, ssp


def kernel(xn, xe, edge_index, K1Nopen, K2Nopen, K1Eopen, K2Eopen, KNclose,
           KE1, KE2, KN1, KN2):
    xn_r = xn[0].T
    xe64 = xe[0].T.reshape(E4, 64)
    src3d = edge_index[0].reshape(SC_NW, NCH, CH)
    dst3d = edge_index[1].reshape(SC_NW, NCH, CH)
    zero = jnp.zeros((N, C), jnp.float32)

    i4 = jnp.eye(4, dtype=jnp.float32)
    A, B, Cm = KE1[:, :, 0:C], KE1[:, :, C:2 * C], KE1[:, :, 2 * C:3 * C]
    WP = jnp.transpose(0.5 * A + Cm, (0, 2, 1))
    WQ = jnp.transpose(0.5 * A - Cm, (0, 2, 1))
    WBbd = jnp.stack([jnp.kron(i4, jnp.transpose(B[i])) for i in range(NLAYER)])
    KE2bd = jnp.stack([jnp.kron(i4, jnp.transpose(KE2[i]))
                       for i in range(NLAYER)])
    D, F, Gm = KN1[:, :, 0:C], KN1[:, :, C:2 * C], KN1[:, :, 2 * C:3 * C]
    WSD = jnp.transpose(0.5 * D + F, (0, 2, 1))
    WSS = jnp.transpose(0.5 * D - F, (0, 2, 1))
    WX = jnp.transpose(Gm, (0, 2, 1))
    KN2T = jnp.transpose(KN2, (0, 2, 1))
    KNcloseT = KNclose.T

    xn_r, p, q = _node_open(xn_r, K1Nopen.T, K2Nopen.T, WP[0], WQ[0])
    xe128 = _edge_open(xe64, jnp.kron(i4, K1Eopen.T), jnp.kron(i4, K2Eopen.T))

    for i in range(NLAYER):
        g, sdp, ssp = _graph_ops(p, q, xe128, dst3d, src3d, zero)
        xe128 = _edge_path(g, xe128, WBbd[i], KE2bd[i])
        if i + 1 < NLAYER:
            wp_n, wq_n = WP[i + 1], WQ[i + 1]
        else:
            wp_n, wq_n = KNcloseT, KNcloseT
        xn_r, p, q = _node_layer(sdp, ssp, xn_r, WSD[i], WSS[i], WX[i],
                                 KN2T[i], wp_n, wq_n)

    return (p.T[None], xe128.reshape(E, C).T[None])
